# baseline (speedup 1.0000x reference)
import jax
import jax.numpy as jnp
from jax.experimental import pallas as pl
from jax.experimental.pallas import tpu as pltpu

_ROWS = 2048


_SPLIT = 1


def _body(idx_ref, *refs):
    x_refs, o_ref = refs[:-1], refs[-1]
    half = 2048 // _SPLIT
    acc = None
    for k, x_ref in enumerate(x_refs):
        c = jax.lax.broadcasted_iota(jnp.int32, (half, 64), 0) + k * half
        sel = (c == idx_ref[:][None, :]).astype(jnp.float32)
        part = jnp.dot(x_ref[:], sel, preferred_element_type=jnp.float32)
        acc = part if acc is None else acc + part
    o_ref[:] = acc


def kernel(x, indices):
    b, s, ch = x.shape
    rows = b * s
    x2 = x.reshape(rows, ch)
    grid = rows // _ROWS
    half = ch // _SPLIT
    in_specs = [pl.BlockSpec((indices.shape[0],), lambda i: (0,))]
    for k in range(_SPLIT):
        in_specs.append(
            pl.BlockSpec((_ROWS, half), lambda i, _k=k: (i, _k))
        )
    out = pl.pallas_call(
        _body,
        grid=(grid,),
        in_specs=in_specs,
        out_specs=pl.BlockSpec((_ROWS, indices.shape[0]), lambda i: (i, 0)),
        out_shape=jax.ShapeDtypeStruct((rows, indices.shape[0]), x.dtype),
    )(indices, *([x2] * _SPLIT))
    return out.reshape(b, s, indices.shape[0])
